# Initial kernel scaffold; baseline (speedup 1.0000x reference)
import jax
import jax.numpy as jnp
from jax.experimental import pallas as pl


def kernel(x, edge_index, batch, W1, b1, W2, b2, fcW, fcb):
    raise NotImplementedError("write your pallas kernel here")


# speedup vs baseline: 21.0567x; 21.0567x over previous
import functools

import jax
import jax.numpy as jnp
from jax import lax
from jax.experimental import pallas as pl
from jax.experimental.pallas import tpu as pltpu
from jax.experimental.pallas import tpu_sc as plsc

N = 50000
E = 800000
C = 64
G = 8
NPAD = 50176
NROW = 392
EPAD = 802816
NS = 16
CHK = 392
HCHK = CHK // 2
BR = 8
GRID = NROW // BR
F32 = jnp.float32
BF16 = jnp.bfloat16
I32 = jnp.int32

def _zero_vec(ref, nwords):
    @pl.loop(0, nwords // 16)
    def _(i):
        ref[pl.ds(i * 16, 16)] = jnp.zeros((16,), F32)


def _zero_acc_chunks(zbuf, acc_sh, s):
    @pl.loop(0, 25)
    def _(j):
        idx = j * 16 + s

        @pl.when(idx < NROW)
        def _():
            pltpu.sync_copy(zbuf, acc_sh.at[pl.ds(idx * 128, 128)])


def _readout_chunks(acc_sh, out_hbm, c, s):
    @pl.loop(0, 25)
    def _(j):
        idx = j * 16 + s

        @pl.when(idx < NROW)
        def _():
            pltpu.sync_copy(acc_sh.at[pl.ds(idx * 128, 128)],
                            out_hbm.at[pl.ds(c * NPAD + idx * 128, 128)])


@functools.cache
def _sc_kernels():
    mesh = plsc.VectorSubcoreMesh(core_axis_name="c", subcore_axis_name="s")
    params = pltpu.CompilerParams(use_tc_tiling_on_sc=False)

    deg = functools.partial(
        pl.kernel,
        out_type=jax.ShapeDtypeStruct((2 * NPAD,), F32),
        mesh=mesh,
        scratch_types=[
            pltpu.VMEM((128,), F32),
            pltpu.VMEM((128,), F32),
            pltpu.VMEM((HCHK, 128), I32),
            pltpu.VMEM_SHARED((NPAD,), F32),
        ],
    )(_deg_sc)
    agg1 = functools.partial(
        pl.kernel,
        out_type=jax.ShapeDtypeStruct((2 * NPAD,), F32),
        mesh=mesh,
        scratch_types=[
            pltpu.VMEM((HCHK, 128), I32),
            pltpu.VMEM((HCHK, 128), I32),
            pltpu.VMEM((128,), F32),
            pltpu.VMEM((128,), F32),
            pltpu.VMEM_SHARED((NPAD,), F32),
            pltpu.VMEM_SHARED((NPAD,), F32),
            pltpu.SemaphoreType.DMA,
        ],
    )(_agg1_sc)
    agg2 = functools.partial(
        pl.kernel,
        out_type=jax.ShapeDtypeStruct((4 * NPAD, 16), BF16),
        mesh=mesh,
        scratch_types=[
            pltpu.VMEM((CHK, 128), I32),
            pltpu.VMEM((CHK, 128), I32),
            pltpu.VMEM((128, 16), BF16),
            pltpu.VMEM((128, 16), BF16),
            pltpu.VMEM_SHARED((NPAD, 16), BF16),
            pltpu.SemaphoreType.DMA,
        ],
        compiler_params=params,
    )(_agg2_sc)
    return deg, agg1, agg2


def _deg_sc(col_hbm, out_hbm, ones_v, zbuf, colbuf, acc_sh):
    c = lax.axis_index("c")
    s = lax.axis_index("s")
    w = s * 2 + c

    @pl.loop(0, 8)
    def _(i):
        ones_v[pl.ds(i * 16, 16)] = jnp.ones((16,), F32)

    _zero_vec(zbuf, 128)
    _zero_acc_chunks(zbuf, acc_sh, s)
    plsc.subcore_barrier()

    pltpu.sync_copy(col_hbm.at[w], colbuf)

    @pl.loop(0, HCHK)
    def _(k):
        pltpu.sync_copy(ones_v, acc_sh.at[colbuf.at[k]], add=True)

    plsc.subcore_barrier()
    _readout_chunks(acc_sh, out_hbm, c, s)


def _prep_tc(degp_ref, x_ref, dinv_ref, u_ref):
    deg = degp_ref[0] + degp_ref[1] + 1.0
    dinv = lax.rsqrt(deg)
    dinv_ref[...] = dinv
    u_ref[...] = x_ref[...] * dinv


def _agg1_sc(row_hbm, col_hbm, u_hbm, out_hbm,
             rowbuf, colbuf, vbuf, zbuf, u_sh, acc_sh, sem):
    c = lax.axis_index("c")
    s = lax.axis_index("s")
    w = s * 2 + c

    _zero_vec(zbuf, 128)
    _zero_acc_chunks(zbuf, acc_sh, s)

    @pl.loop(0, 25)
    def _(j):
        idx = j * 16 + s

        @pl.when(idx < NROW)
        def _():
            pltpu.sync_copy(u_hbm.at[pl.ds(idx * 128, 128)],
                            u_sh.at[pl.ds(idx * 128, 128)])

    plsc.subcore_barrier()

    pltpu.sync_copy(row_hbm.at[w], rowbuf)
    pltpu.sync_copy(col_hbm.at[w], colbuf)

    @pl.loop(0, HCHK)
    def _(k):
        pltpu.async_copy(u_sh.at[rowbuf.at[k]], vbuf, sem).wait()
        pltpu.sync_copy(vbuf, acc_sh.at[colbuf.at[k]], add=True)

    plsc.subcore_barrier()
    _readout_chunks(acc_sh, out_hbm, c, s)


def _h1_tc(x_ref, t_ref, dinv_ref, w1_ref, b1_ref, h1_ref, g_ref):
    dinv = dinv_ref[...]
    s1 = dinv * (t_ref[0] + t_ref[1] + x_ref[...] * dinv)
    s1t = jnp.transpose(s1)
    xt = jnp.transpose(x_ref[...])
    dt = jnp.transpose(dinv)
    w1 = w1_ref[...]
    b1 = b1_ref[...]
    for u in range(BR):
        s1c = s1t[:, u:u + 1]
        z = s1c * w1 + b1
        h1 = xt[:, u:u + 1] + z * jax.nn.sigmoid(z)
        g = h1 * dt[:, u:u + 1]
        h1_ref[pl.ds(u * 128, 128), :] = h1
        gb = g.astype(BF16)
        for p in range(4):
            g_ref[p, pl.ds(u * 128, 128), :] = gb[:, p * 16:(p + 1) * 16]


def _agg2_sc(rowq_hbm, col_hbm, g_hbm, out_hbm,
             rowbuf, colbuf, gbuf, zbuf, acc_sh, sem):
    c = lax.axis_index("c")
    s = lax.axis_index("s")

    @pl.loop(0, 64)
    def _(r):
        zbuf[pl.ds(r * 2, 2), :] = jnp.zeros((2, 16), BF16)

    pltpu.sync_copy(col_hbm.at[s], colbuf)

    for q in range(2):
        p = 2 * q + c
        _zero_acc_chunks(zbuf, acc_sh, s)
        plsc.subcore_barrier()

        pltpu.sync_copy(rowq_hbm.at[p, s], rowbuf)

        @pl.loop(0, CHK)
        def _(k):
            pltpu.async_copy(g_hbm.at[rowbuf.at[k]], gbuf, sem).wait()
            pltpu.sync_copy(gbuf, acc_sh.at[colbuf.at[k]], add=True)

        plsc.subcore_barrier()

        @pl.loop(0, 25)
        def _(j):
            idx = j * 16 + s

            @pl.when(idx < NROW)
            def _():
                pltpu.sync_copy(acc_sh.at[pl.ds(idx * 128, 128)],
                                out_hbm.at[pl.ds(p * NPAD + idx * 128, 128)])

        plsc.subcore_barrier()


def _fin_tc(acc_ref, h1_ref, dinv_ref, batch_ref, w2_ref, b2_ref,
            fcw_ref, fcb_ref, out_ref, a2s, masks, psum, pcnt):
    i = pl.program_id(0)

    @pl.when(i == 0)
    def _():
        psum[...] = jnp.zeros((G, C), F32)
        pcnt[...] = jnp.zeros((1, G), F32)

    dt = jnp.transpose(dinv_ref[...])
    bt = jnp.transpose(batch_ref[...])
    h1 = h1_ref[...]
    gid = lax.broadcasted_iota(I32, (128, G), 1)
    for u in range(BR):
        dc = dt[:, u:u + 1]
        accu = jnp.concatenate([acc_ref[p, pl.ds(u * 128, 128), :]
                                for p in range(4)], axis=1).astype(F32)
        h1u = h1[u * 128:(u + 1) * 128, :]
        a2s[pl.ds(u * 128, 128), :] = dc * accu + (dc * dc) * h1u
        masks[pl.ds(u * 128, 128), :] = (bt[:, u:u + 1] == gid).astype(F32)

    out2 = jnp.dot(a2s[...], w2_ref[...], preferred_element_type=F32)
    out2 = out2 + b2_ref[...]
    h2 = h1 + out2 * jax.nn.sigmoid(out2)
    m = masks[...]
    psum[...] += lax.dot_general(m, h2, (((0,), (0,)), ((), ())),
                                 preferred_element_type=F32)
    pcnt[...] += jnp.sum(m, axis=0, keepdims=True)

    @pl.when(i == GRID - 1)
    def _():
        cnt = jnp.maximum(pcnt[...], 1.0)
        pooled = psum[...] / jnp.transpose(cnt)
        z = jnp.dot(pooled, fcw_ref[...], preferred_element_type=F32)
        out_ref[...] = jax.nn.sigmoid(z + fcb_ref[...])


def kernel(x, edge_index, batch, W1, b1, W2, b2, fcW, fcb):
    ei = edge_index.astype(I32)
    rowf = jnp.concatenate([ei[0], jnp.zeros((EPAD - E,), I32)])
    colf = jnp.concatenate([ei[1], jnp.full((EPAD - E,), N, I32)])
    row_t = rowf.reshape(NS, CHK, 128)
    col_t = colf.reshape(NS, CHK, 128)
    row_w = rowf.reshape(2 * NS, HCHK, 128)
    col_w = colf.reshape(2 * NS, HCHK, 128)
    rowq = (row_t[None] +
            (NPAD * jnp.arange(4, dtype=I32))[:, None, None, None])
    x2 = jnp.concatenate([x[:, 0], jnp.zeros((NPAD - N,), F32)]).reshape(NROW, 128)
    batch2 = jnp.concatenate([batch.astype(I32),
                              jnp.full((NPAD - N,), 127, I32)]).reshape(NROW, 128)

    deg_k, agg1_k, agg2_k = _sc_kernels()
    degp = deg_k(col_w).reshape(2, NROW, 128)

    dinv2, u2 = pl.pallas_call(
        _prep_tc,
        out_shape=[jax.ShapeDtypeStruct((NROW, 128), F32),
                   jax.ShapeDtypeStruct((NROW, 128), F32)],
    )(degp, x2)

    t = agg1_k(row_w, col_w, u2.reshape(NPAD)).reshape(2, NROW, 128)

    h1, gpair = pl.pallas_call(
        _h1_tc,
        grid=(GRID,),
        in_specs=[
            pl.BlockSpec((BR, 128), lambda i: (i, 0)),
            pl.BlockSpec((2, BR, 128), lambda i: (0, i, 0)),
            pl.BlockSpec((BR, 128), lambda i: (i, 0)),
            pl.BlockSpec((1, C), lambda i: (0, 0)),
            pl.BlockSpec((1, C), lambda i: (0, 0)),
        ],
        out_specs=[
            pl.BlockSpec((BR * 128, C), lambda i: (i, 0)),
            pl.BlockSpec((4, BR * 128, 16), lambda i: (0, i, 0)),
        ],
        out_shape=[jax.ShapeDtypeStruct((NPAD, C), F32),
                   jax.ShapeDtypeStruct((4, NPAD, 16), BF16)],
    )(x2, t, dinv2, W1, b1.reshape(1, C))

    acc = agg2_k(rowq, col_t, gpair.reshape(4 * NPAD, 16))

    out = pl.pallas_call(
        _fin_tc,
        grid=(GRID,),
        in_specs=[
            pl.BlockSpec((4, BR * 128, 16), lambda i: (0, i, 0)),
            pl.BlockSpec((BR * 128, C), lambda i: (i, 0)),
            pl.BlockSpec((BR, 128), lambda i: (i, 0)),
            pl.BlockSpec((BR, 128), lambda i: (i, 0)),
            pl.BlockSpec((C, C), lambda i: (0, 0)),
            pl.BlockSpec((1, C), lambda i: (0, 0)),
            pl.BlockSpec((C, 1), lambda i: (0, 0)),
            pl.BlockSpec((1, 1), lambda i: (0, 0)),
        ],
        out_specs=pl.BlockSpec((G, 1), lambda i: (0, 0)),
        out_shape=jax.ShapeDtypeStruct((G, 1), F32),
        scratch_shapes=[
            pltpu.VMEM((BR * 128, C), F32),
            pltpu.VMEM((BR * 128, G), F32),
            pltpu.VMEM((G, C), F32),
            pltpu.VMEM((1, G), F32),
        ],
    )(acc.reshape(4, NPAD, 16),
      h1, dinv2, batch2, W2, b2.reshape(1, C), fcW, fcb.reshape(1, 1))

    return out


# speedup vs baseline: 23.1846x; 1.1011x over previous
import functools

import jax
import jax.numpy as jnp
from jax import lax
from jax.experimental import pallas as pl
from jax.experimental.pallas import tpu as pltpu
from jax.experimental.pallas import tpu_sc as plsc

N = 50000
E = 800000
C = 64
G = 8
NPAD = 50176
NROW = 392
EPAD = 802816
NS = 16
CHK = 392
HCHK = CHK // 2
BR = 8
GRID = NROW // BR
F32 = jnp.float32
BF16 = jnp.bfloat16
I32 = jnp.int32

def _zero_vec(ref, nwords):
    @pl.loop(0, nwords // 16)
    def _(i):
        ref[pl.ds(i * 16, 16)] = jnp.zeros((16,), F32)


def _zero_acc_chunks(zbuf, acc_sh, s):
    @pl.loop(0, 25)
    def _(j):
        idx = j * 16 + s

        @pl.when(idx < NROW)
        def _():
            pltpu.sync_copy(zbuf, acc_sh.at[pl.ds(idx * 128, 128)])


def _readout_chunks(acc_sh, out_hbm, c, s):
    @pl.loop(0, 25)
    def _(j):
        idx = j * 16 + s

        @pl.when(idx < NROW)
        def _():
            pltpu.sync_copy(acc_sh.at[pl.ds(idx * 128, 128)],
                            out_hbm.at[pl.ds(c * NPAD + idx * 128, 128)])


@functools.cache
def _sc_kernels():
    mesh = plsc.VectorSubcoreMesh(core_axis_name="c", subcore_axis_name="s")
    params = pltpu.CompilerParams(use_tc_tiling_on_sc=False)

    deg = functools.partial(
        pl.kernel,
        out_type=jax.ShapeDtypeStruct((2 * NPAD,), F32),
        mesh=mesh,
        scratch_types=[
            pltpu.VMEM((128,), F32),
            pltpu.VMEM((128,), F32),
            pltpu.VMEM((HCHK, 128), I32),
            pltpu.VMEM_SHARED((NPAD,), F32),
            pltpu.SemaphoreType.DMA,
        ],
    )(_deg_sc)
    agg1 = functools.partial(
        pl.kernel,
        out_type=jax.ShapeDtypeStruct((2 * NPAD,), F32),
        mesh=mesh,
        scratch_types=[
            pltpu.VMEM((HCHK, 128), I32),
            pltpu.VMEM((HCHK, 128), I32),
            pltpu.VMEM((128,), F32),
            pltpu.VMEM((128,), F32),
            pltpu.VMEM((128,), F32),
            pltpu.VMEM_SHARED((NPAD,), F32),
            pltpu.VMEM_SHARED((NPAD,), F32),
            pltpu.SemaphoreType.DMA,
            pltpu.SemaphoreType.DMA,
        ],
    )(_agg1_sc)
    agg2 = functools.partial(
        pl.kernel,
        out_type=jax.ShapeDtypeStruct((4 * NPAD, 16), BF16),
        mesh=mesh,
        scratch_types=[
            pltpu.VMEM((CHK, 128), I32),
            pltpu.VMEM((CHK, 128), I32),
            pltpu.VMEM((128, 16), BF16),
            pltpu.VMEM((128, 16), BF16),
            pltpu.VMEM((128, 16), BF16),
            pltpu.VMEM_SHARED((NPAD, 16), BF16),
            pltpu.SemaphoreType.DMA,
            pltpu.SemaphoreType.DMA,
        ],
        compiler_params=params,
    )(_agg2_sc)
    return deg, agg1, agg2


def _deg_sc(col_hbm, out_hbm, ones_v, zbuf, colbuf, acc_sh, ssem):
    c = lax.axis_index("c")
    s = lax.axis_index("s")
    w = s * 2 + c

    @pl.loop(0, 8)
    def _(i):
        ones_v[pl.ds(i * 16, 16)] = jnp.ones((16,), F32)

    _zero_vec(zbuf, 128)
    _zero_acc_chunks(zbuf, acc_sh, s)
    plsc.subcore_barrier()

    pltpu.sync_copy(col_hbm.at[w], colbuf)

    @pl.loop(0, HCHK // 4)
    def _(m):
        for j in range(4):
            pltpu.async_copy(ones_v, acc_sh.at[colbuf.at[m * 4 + j]], ssem,
                             add=True)
        for j in range(4):
            pltpu.make_async_copy(ones_v, acc_sh.at[colbuf.at[m * 4 + j]],
                                  ssem).wait()

    plsc.subcore_barrier()
    _readout_chunks(acc_sh, out_hbm, c, s)


def _prep_tc(degp_ref, x_ref, dinv_ref, u_ref):
    deg = degp_ref[0] + degp_ref[1] + 1.0
    dinv = lax.rsqrt(deg)
    dinv_ref[...] = dinv
    u_ref[...] = x_ref[...] * dinv


def _agg1_sc(row_hbm, col_hbm, u_hbm, out_hbm,
             rowbuf, colbuf, vbuf0, vbuf1, zbuf, u_sh, acc_sh, sem0, sem1):
    c = lax.axis_index("c")
    s = lax.axis_index("s")
    w = s * 2 + c

    _zero_vec(zbuf, 128)
    _zero_acc_chunks(zbuf, acc_sh, s)

    @pl.loop(0, 25)
    def _(j):
        idx = j * 16 + s

        @pl.when(idx < NROW)
        def _():
            pltpu.sync_copy(u_hbm.at[pl.ds(idx * 128, 128)],
                            u_sh.at[pl.ds(idx * 128, 128)])

    plsc.subcore_barrier()

    pltpu.sync_copy(row_hbm.at[w], rowbuf)
    pltpu.sync_copy(col_hbm.at[w], colbuf)

    pltpu.async_copy(u_sh.at[rowbuf.at[0]], vbuf0, sem0)

    @pl.loop(0, HCHK // 2)
    def _(m):
        k0 = m * 2
        pltpu.make_async_copy(u_sh.at[rowbuf.at[k0]], vbuf0, sem0).wait()
        pltpu.async_copy(u_sh.at[rowbuf.at[k0 + 1]], vbuf1, sem1)
        pltpu.sync_copy(vbuf0, acc_sh.at[colbuf.at[k0]], add=True)
        pltpu.make_async_copy(u_sh.at[rowbuf.at[k0 + 1]], vbuf1, sem1).wait()

        @pl.when(k0 + 2 < HCHK)
        def _():
            pltpu.async_copy(u_sh.at[rowbuf.at[k0 + 2]], vbuf0, sem0)

        pltpu.sync_copy(vbuf1, acc_sh.at[colbuf.at[k0 + 1]], add=True)

    plsc.subcore_barrier()
    _readout_chunks(acc_sh, out_hbm, c, s)


def _h1_tc(x_ref, t_ref, dinv_ref, w1_ref, b1_ref, h1_ref, g_ref):
    dinv = dinv_ref[...]
    s1 = dinv * (t_ref[0] + t_ref[1] + x_ref[...] * dinv)
    s1t = jnp.transpose(s1)
    xt = jnp.transpose(x_ref[...])
    dt = jnp.transpose(dinv)
    w1 = w1_ref[...]
    b1 = b1_ref[...]
    for u in range(BR):
        s1c = s1t[:, u:u + 1]
        z = s1c * w1 + b1
        h1 = xt[:, u:u + 1] + z * jax.nn.sigmoid(z)
        g = h1 * dt[:, u:u + 1]
        h1_ref[pl.ds(u * 128, 128), :] = h1
        gb = g.astype(BF16)
        for p in range(4):
            g_ref[p, pl.ds(u * 128, 128), :] = gb[:, p * 16:(p + 1) * 16]


def _agg2_sc(rowq_hbm, col_hbm, g_hbm, out_hbm,
             rowbuf, colbuf, gbuf0, gbuf1, zbuf, acc_sh, sem0, sem1):
    c = lax.axis_index("c")
    s = lax.axis_index("s")

    @pl.loop(0, 64)
    def _(r):
        zbuf[pl.ds(r * 2, 2), :] = jnp.zeros((2, 16), BF16)

    pltpu.sync_copy(col_hbm.at[s], colbuf)

    for q in range(2):
        p = 2 * q + c
        _zero_acc_chunks(zbuf, acc_sh, s)
        plsc.subcore_barrier()

        pltpu.sync_copy(rowq_hbm.at[p, s], rowbuf)

        pltpu.async_copy(g_hbm.at[rowbuf.at[0]], gbuf0, sem0)

        @pl.loop(0, CHK // 2)
        def _(m):
            k0 = m * 2
            pltpu.make_async_copy(g_hbm.at[rowbuf.at[k0]], gbuf0, sem0).wait()
            pltpu.async_copy(g_hbm.at[rowbuf.at[k0 + 1]], gbuf1, sem1)
            pltpu.sync_copy(gbuf0, acc_sh.at[colbuf.at[k0]], add=True)
            pltpu.make_async_copy(g_hbm.at[rowbuf.at[k0 + 1]], gbuf1,
                                  sem1).wait()

            @pl.when(k0 + 2 < CHK)
            def _():
                pltpu.async_copy(g_hbm.at[rowbuf.at[k0 + 2]], gbuf0, sem0)

            pltpu.sync_copy(gbuf1, acc_sh.at[colbuf.at[k0 + 1]], add=True)

        plsc.subcore_barrier()

        @pl.loop(0, 25)
        def _(j):
            idx = j * 16 + s

            @pl.when(idx < NROW)
            def _():
                pltpu.sync_copy(acc_sh.at[pl.ds(idx * 128, 128)],
                                out_hbm.at[pl.ds(p * NPAD + idx * 128, 128)])

        plsc.subcore_barrier()


def _fin_tc(acc_ref, h1_ref, dinv_ref, batch_ref, w2_ref, b2_ref,
            fcw_ref, fcb_ref, out_ref, a2s, masks, psum, pcnt):
    i = pl.program_id(0)

    @pl.when(i == 0)
    def _():
        psum[...] = jnp.zeros((G, C), F32)
        pcnt[...] = jnp.zeros((1, G), F32)

    dt = jnp.transpose(dinv_ref[...])
    bt = jnp.transpose(batch_ref[...])
    h1 = h1_ref[...]
    gid = lax.broadcasted_iota(I32, (128, G), 1)
    for u in range(BR):
        dc = dt[:, u:u + 1]
        accu = jnp.concatenate([acc_ref[p, pl.ds(u * 128, 128), :]
                                for p in range(4)], axis=1).astype(F32)
        h1u = h1[u * 128:(u + 1) * 128, :]
        a2s[pl.ds(u * 128, 128), :] = dc * accu + (dc * dc) * h1u
        masks[pl.ds(u * 128, 128), :] = (bt[:, u:u + 1] == gid).astype(F32)

    out2 = jnp.dot(a2s[...], w2_ref[...], preferred_element_type=F32)
    out2 = out2 + b2_ref[...]
    h2 = h1 + out2 * jax.nn.sigmoid(out2)
    m = masks[...]
    psum[...] += lax.dot_general(m, h2, (((0,), (0,)), ((), ())),
                                 preferred_element_type=F32)
    pcnt[...] += jnp.sum(m, axis=0, keepdims=True)

    @pl.when(i == GRID - 1)
    def _():
        cnt = jnp.maximum(pcnt[...], 1.0)
        pooled = psum[...] / jnp.transpose(cnt)
        z = jnp.dot(pooled, fcw_ref[...], preferred_element_type=F32)
        out_ref[...] = jax.nn.sigmoid(z + fcb_ref[...])


def kernel(x, edge_index, batch, W1, b1, W2, b2, fcW, fcb):
    ei = edge_index.astype(I32)
    rowf = jnp.concatenate([ei[0], jnp.zeros((EPAD - E,), I32)])
    colf = jnp.concatenate([ei[1], jnp.full((EPAD - E,), N, I32)])
    row_t = rowf.reshape(NS, CHK, 128)
    col_t = colf.reshape(NS, CHK, 128)
    row_w = rowf.reshape(2 * NS, HCHK, 128)
    col_w = colf.reshape(2 * NS, HCHK, 128)
    rowq = (row_t[None] +
            (NPAD * jnp.arange(4, dtype=I32))[:, None, None, None])
    x2 = jnp.concatenate([x[:, 0], jnp.zeros((NPAD - N,), F32)]).reshape(NROW, 128)
    batch2 = jnp.concatenate([batch.astype(I32),
                              jnp.full((NPAD - N,), 127, I32)]).reshape(NROW, 128)

    deg_k, agg1_k, agg2_k = _sc_kernels()
    degp = deg_k(col_w).reshape(2, NROW, 128)

    dinv2, u2 = pl.pallas_call(
        _prep_tc,
        out_shape=[jax.ShapeDtypeStruct((NROW, 128), F32),
                   jax.ShapeDtypeStruct((NROW, 128), F32)],
    )(degp, x2)

    t = agg1_k(row_w, col_w, u2.reshape(NPAD)).reshape(2, NROW, 128)

    h1, gpair = pl.pallas_call(
        _h1_tc,
        grid=(GRID,),
        in_specs=[
            pl.BlockSpec((BR, 128), lambda i: (i, 0)),
            pl.BlockSpec((2, BR, 128), lambda i: (0, i, 0)),
            pl.BlockSpec((BR, 128), lambda i: (i, 0)),
            pl.BlockSpec((1, C), lambda i: (0, 0)),
            pl.BlockSpec((1, C), lambda i: (0, 0)),
        ],
        out_specs=[
            pl.BlockSpec((BR * 128, C), lambda i: (i, 0)),
            pl.BlockSpec((4, BR * 128, 16), lambda i: (0, i, 0)),
        ],
        out_shape=[jax.ShapeDtypeStruct((NPAD, C), F32),
                   jax.ShapeDtypeStruct((4, NPAD, 16), BF16)],
    )(x2, t, dinv2, W1, b1.reshape(1, C))

    acc = agg2_k(rowq, col_t, gpair.reshape(4 * NPAD, 16))

    out = pl.pallas_call(
        _fin_tc,
        grid=(GRID,),
        in_specs=[
            pl.BlockSpec((4, BR * 128, 16), lambda i: (0, i, 0)),
            pl.BlockSpec((BR * 128, C), lambda i: (i, 0)),
            pl.BlockSpec((BR, 128), lambda i: (i, 0)),
            pl.BlockSpec((BR, 128), lambda i: (i, 0)),
            pl.BlockSpec((C, C), lambda i: (0, 0)),
            pl.BlockSpec((1, C), lambda i: (0, 0)),
            pl.BlockSpec((C, 1), lambda i: (0, 0)),
            pl.BlockSpec((1, 1), lambda i: (0, 0)),
        ],
        out_specs=pl.BlockSpec((G, 1), lambda i: (0, 0)),
        out_shape=jax.ShapeDtypeStruct((G, 1), F32),
        scratch_shapes=[
            pltpu.VMEM((BR * 128, C), F32),
            pltpu.VMEM((BR * 128, G), F32),
            pltpu.VMEM((G, C), F32),
            pltpu.VMEM((1, G), F32),
        ],
    )(acc.reshape(4, NPAD, 16),
      h1, dinv2, batch2, W2, b2.reshape(1, C), fcW, fcb.reshape(1, 1))

    return out


# speedup vs baseline: 35.1933x; 1.5180x over previous
import functools

import jax
import jax.numpy as jnp
from jax import lax
from jax.experimental import pallas as pl
from jax.experimental.pallas import tpu as pltpu
from jax.experimental.pallas import tpu_sc as plsc

N = 50000
E = 800000
C = 64
G = 8
NPAD = 50176
NROW = 392
EPAD = 802816
NS = 16
CHK = 392
HCHK = CHK // 2
BR = 8
GRID = NROW // BR
F32 = jnp.float32
BF16 = jnp.bfloat16
I32 = jnp.int32

def _zero_vec(ref, nwords):
    @pl.loop(0, nwords // 16)
    def _(i):
        ref[pl.ds(i * 16, 16)] = jnp.zeros((16,), F32)


def _zero_acc_chunks(zbuf, acc_sh, s):
    @pl.loop(0, 25)
    def _(j):
        idx = j * 16 + s

        @pl.when(idx < NROW)
        def _():
            pltpu.sync_copy(zbuf, acc_sh.at[pl.ds(idx * 128, 128)])


def _readout_chunks(acc_sh, out_hbm, c, s):
    @pl.loop(0, 25)
    def _(j):
        idx = j * 16 + s

        @pl.when(idx < NROW)
        def _():
            pltpu.sync_copy(acc_sh.at[pl.ds(idx * 128, 128)],
                            out_hbm.at[pl.ds(c * NPAD + idx * 128, 128)])


@functools.cache
def _sc_kernels():
    mesh = plsc.VectorSubcoreMesh(core_axis_name="c", subcore_axis_name="s")
    params = pltpu.CompilerParams(use_tc_tiling_on_sc=False)

    deg = functools.partial(
        pl.kernel,
        out_type=jax.ShapeDtypeStruct((2 * NPAD,), F32),
        mesh=mesh,
        scratch_types=[
            pltpu.VMEM((128,), F32),
            pltpu.VMEM((128,), F32),
            pltpu.VMEM((HCHK, 128), I32),
            pltpu.VMEM_SHARED((NPAD,), F32),
            pltpu.SemaphoreType.DMA,
        ],
    )(_deg_sc)
    agg1 = functools.partial(
        pl.kernel,
        out_type=jax.ShapeDtypeStruct((2 * NPAD,), F32),
        mesh=mesh,
        scratch_types=[
            pltpu.VMEM((HCHK, 128), I32),
            pltpu.VMEM((HCHK, 128), I32),
            pltpu.VMEM((128,), F32),
            pltpu.VMEM((128,), F32),
            pltpu.VMEM((128,), F32),
            pltpu.VMEM_SHARED((NPAD,), F32),
            pltpu.VMEM_SHARED((NPAD,), F32),
            pltpu.SemaphoreType.DMA,
            pltpu.SemaphoreType.DMA,
        ],
    )(_agg1_sc)
    agg2 = functools.partial(
        pl.kernel,
        out_type=jax.ShapeDtypeStruct((4 * NPAD, 16), BF16),
        mesh=mesh,
        scratch_types=[
            pltpu.VMEM((CHK, 128), I32),
            pltpu.VMEM((CHK, 128), I32),
            pltpu.VMEM((4, 128, 16), BF16),
            pltpu.VMEM((128, 16), BF16),
            pltpu.VMEM_SHARED((NPAD, 16), BF16),
            pltpu.SemaphoreType.DMA((4,)),
            pltpu.SemaphoreType.DMA((4,)),
        ],
        compiler_params=params,
    )(_agg2_sc)
    return deg, agg1, agg2


def _deg_sc(col_hbm, out_hbm, ones_v, zbuf, colbuf, acc_sh, ssem):
    c = lax.axis_index("c")
    s = lax.axis_index("s")
    w = s * 2 + c

    @pl.loop(0, 8)
    def _(i):
        ones_v[pl.ds(i * 16, 16)] = jnp.ones((16,), F32)

    _zero_vec(zbuf, 128)
    _zero_acc_chunks(zbuf, acc_sh, s)
    plsc.subcore_barrier()

    pltpu.sync_copy(col_hbm.at[w], colbuf)

    @pl.loop(0, HCHK // 4)
    def _(m):
        for j in range(4):
            pltpu.async_copy(ones_v, acc_sh.at[colbuf.at[m * 4 + j]], ssem,
                             add=True)
        for j in range(4):
            pltpu.make_async_copy(ones_v, acc_sh.at[colbuf.at[m * 4 + j]],
                                  ssem).wait()

    plsc.subcore_barrier()
    _readout_chunks(acc_sh, out_hbm, c, s)


def _prep_tc(degp_ref, x_ref, dinv_ref, u_ref):
    deg = degp_ref[0] + degp_ref[1] + 1.0
    dinv = lax.rsqrt(deg)
    dinv_ref[...] = dinv
    u_ref[...] = x_ref[...] * dinv


def _agg1_sc(row_hbm, col_hbm, u_hbm, out_hbm,
             rowbuf, colbuf, vbuf0, vbuf1, zbuf, u_sh, acc_sh, sem0, sem1):
    c = lax.axis_index("c")
    s = lax.axis_index("s")
    w = s * 2 + c

    _zero_vec(zbuf, 128)
    _zero_acc_chunks(zbuf, acc_sh, s)

    @pl.loop(0, 25)
    def _(j):
        idx = j * 16 + s

        @pl.when(idx < NROW)
        def _():
            pltpu.sync_copy(u_hbm.at[pl.ds(idx * 128, 128)],
                            u_sh.at[pl.ds(idx * 128, 128)])

    plsc.subcore_barrier()

    pltpu.sync_copy(row_hbm.at[w], rowbuf)
    pltpu.sync_copy(col_hbm.at[w], colbuf)

    pltpu.async_copy(u_sh.at[rowbuf.at[0]], vbuf0, sem0)

    @pl.loop(0, HCHK // 2)
    def _(m):
        k0 = m * 2
        pltpu.make_async_copy(u_sh.at[rowbuf.at[k0]], vbuf0, sem0).wait()
        pltpu.async_copy(u_sh.at[rowbuf.at[k0 + 1]], vbuf1, sem1)
        pltpu.sync_copy(vbuf0, acc_sh.at[colbuf.at[k0]], add=True)
        pltpu.make_async_copy(u_sh.at[rowbuf.at[k0 + 1]], vbuf1, sem1).wait()

        @pl.when(k0 + 2 < HCHK)
        def _():
            pltpu.async_copy(u_sh.at[rowbuf.at[k0 + 2]], vbuf0, sem0)

        pltpu.sync_copy(vbuf1, acc_sh.at[colbuf.at[k0 + 1]], add=True)

    plsc.subcore_barrier()
    _readout_chunks(acc_sh, out_hbm, c, s)


def _h1_tc(x_ref, t_ref, dinv_ref, w1_ref, b1_ref, h1_ref, g_ref):
    dinv = dinv_ref[...]
    s1 = dinv * (t_ref[0] + t_ref[1] + x_ref[...] * dinv)
    s1t = jnp.transpose(s1)
    xt = jnp.transpose(x_ref[...])
    dt = jnp.transpose(dinv)
    w1 = w1_ref[...]
    b1 = b1_ref[...]
    for u in range(BR):
        s1c = s1t[:, u:u + 1]
        z = s1c * w1 + b1
        h1 = xt[:, u:u + 1] + z * jax.nn.sigmoid(z)
        g = h1 * dt[:, u:u + 1]
        h1_ref[pl.ds(u * 128, 128), :] = h1
        gb = g.astype(BF16)
        for p in range(4):
            g_ref[p, pl.ds(u * 128, 128), :] = gb[:, p * 16:(p + 1) * 16]


def _agg2_sc(rowq_hbm, col_hbm, g_hbm, out_hbm,
             rowbuf, colbuf, gring, zbuf, acc_sh, gsem, ssem):
    c = lax.axis_index("c")
    s = lax.axis_index("s")

    @pl.loop(0, 64)
    def _(r):
        zbuf[pl.ds(r * 2, 2), :] = jnp.zeros((2, 16), BF16)

    pltpu.sync_copy(col_hbm.at[s], colbuf)

    for q in range(2):
        p = 2 * q + c
        _zero_acc_chunks(zbuf, acc_sh, s)
        plsc.subcore_barrier()

        pltpu.sync_copy(rowq_hbm.at[p, s], rowbuf)

        @pl.loop(0, CHK // 4)
        def _(m):
            k = m * 4
            for j in range(4):
                @pl.when(m > 0)
                def _():
                    pltpu.make_async_copy(
                        gring.at[j], acc_sh.at[colbuf.at[k + j - 4]],
                        ssem.at[j]).wait()

                pltpu.async_copy(g_hbm.at[rowbuf.at[k + j]], gring.at[j],
                                 gsem.at[j])
            for j in range(4):
                pltpu.make_async_copy(g_hbm.at[rowbuf.at[k + j]], gring.at[j],
                                      gsem.at[j]).wait()
                pltpu.async_copy(gring.at[j], acc_sh.at[colbuf.at[k + j]],
                                ssem.at[j], add=True)

        for j in range(4):
            pltpu.make_async_copy(gring.at[j],
                                  acc_sh.at[colbuf.at[CHK - 4 + j]],
                                  ssem.at[j]).wait()

        plsc.subcore_barrier()

        @pl.loop(0, 25)
        def _(j):
            idx = j * 16 + s

            @pl.when(idx < NROW)
            def _():
                pltpu.sync_copy(acc_sh.at[pl.ds(idx * 128, 128)],
                                out_hbm.at[pl.ds(p * NPAD + idx * 128, 128)])

        plsc.subcore_barrier()


def _fin_tc(acc_ref, h1_ref, dinv_ref, batch_ref, w2_ref, b2_ref,
            fcw_ref, fcb_ref, out_ref, a2s, masks, psum, pcnt):
    i = pl.program_id(0)

    @pl.when(i == 0)
    def _():
        psum[...] = jnp.zeros((G, C), F32)
        pcnt[...] = jnp.zeros((1, G), F32)

    dt = jnp.transpose(dinv_ref[...])
    bt = jnp.transpose(batch_ref[...])
    h1 = h1_ref[...]
    gid = lax.broadcasted_iota(I32, (128, G), 1)
    for u in range(BR):
        dc = dt[:, u:u + 1]
        accu = jnp.concatenate([acc_ref[p, pl.ds(u * 128, 128), :]
                                for p in range(4)], axis=1).astype(F32)
        h1u = h1[u * 128:(u + 1) * 128, :]
        a2s[pl.ds(u * 128, 128), :] = dc * accu + (dc * dc) * h1u
        masks[pl.ds(u * 128, 128), :] = (bt[:, u:u + 1] == gid).astype(F32)

    out2 = jnp.dot(a2s[...], w2_ref[...], preferred_element_type=F32)
    out2 = out2 + b2_ref[...]
    h2 = h1 + out2 * jax.nn.sigmoid(out2)
    m = masks[...]
    psum[...] += lax.dot_general(m, h2, (((0,), (0,)), ((), ())),
                                 preferred_element_type=F32)
    pcnt[...] += jnp.sum(m, axis=0, keepdims=True)

    @pl.when(i == GRID - 1)
    def _():
        cnt = jnp.maximum(pcnt[...], 1.0)
        pooled = psum[...] / jnp.transpose(cnt)
        z = jnp.dot(pooled, fcw_ref[...], preferred_element_type=F32)
        out_ref[...] = jax.nn.sigmoid(z + fcb_ref[...])


def kernel(x, edge_index, batch, W1, b1, W2, b2, fcW, fcb):
    ei = edge_index.astype(I32)
    rowf = jnp.concatenate([ei[0], jnp.zeros((EPAD - E,), I32)])
    colf = jnp.concatenate([ei[1], jnp.full((EPAD - E,), N, I32)])
    row_t = rowf.reshape(NS, CHK, 128)
    col_t = colf.reshape(NS, CHK, 128)
    row_w = rowf.reshape(2 * NS, HCHK, 128)
    col_w = colf.reshape(2 * NS, HCHK, 128)
    rowq = (row_t[None] +
            (NPAD * jnp.arange(4, dtype=I32))[:, None, None, None])
    x2 = jnp.concatenate([x[:, 0], jnp.zeros((NPAD - N,), F32)]).reshape(NROW, 128)
    batch2 = jnp.concatenate([batch.astype(I32),
                              jnp.full((NPAD - N,), 127, I32)]).reshape(NROW, 128)

    deg_k, agg1_k, agg2_k = _sc_kernels()
    degp = deg_k(col_w).reshape(2, NROW, 128)

    dinv2, u2 = pl.pallas_call(
        _prep_tc,
        out_shape=[jax.ShapeDtypeStruct((NROW, 128), F32),
                   jax.ShapeDtypeStruct((NROW, 128), F32)],
    )(degp, x2)

    t = agg1_k(row_w, col_w, u2.reshape(NPAD)).reshape(2, NROW, 128)

    h1, gpair = pl.pallas_call(
        _h1_tc,
        grid=(GRID,),
        in_specs=[
            pl.BlockSpec((BR, 128), lambda i: (i, 0)),
            pl.BlockSpec((2, BR, 128), lambda i: (0, i, 0)),
            pl.BlockSpec((BR, 128), lambda i: (i, 0)),
            pl.BlockSpec((1, C), lambda i: (0, 0)),
            pl.BlockSpec((1, C), lambda i: (0, 0)),
        ],
        out_specs=[
            pl.BlockSpec((BR * 128, C), lambda i: (i, 0)),
            pl.BlockSpec((4, BR * 128, 16), lambda i: (0, i, 0)),
        ],
        out_shape=[jax.ShapeDtypeStruct((NPAD, C), F32),
                   jax.ShapeDtypeStruct((4, NPAD, 16), BF16)],
    )(x2, t, dinv2, W1, b1.reshape(1, C))

    acc = agg2_k(rowq, col_t, gpair.reshape(4 * NPAD, 16))

    out = pl.pallas_call(
        _fin_tc,
        grid=(GRID,),
        in_specs=[
            pl.BlockSpec((4, BR * 128, 16), lambda i: (0, i, 0)),
            pl.BlockSpec((BR * 128, C), lambda i: (i, 0)),
            pl.BlockSpec((BR, 128), lambda i: (i, 0)),
            pl.BlockSpec((BR, 128), lambda i: (i, 0)),
            pl.BlockSpec((C, C), lambda i: (0, 0)),
            pl.BlockSpec((1, C), lambda i: (0, 0)),
            pl.BlockSpec((C, 1), lambda i: (0, 0)),
            pl.BlockSpec((1, 1), lambda i: (0, 0)),
        ],
        out_specs=pl.BlockSpec((G, 1), lambda i: (0, 0)),
        out_shape=jax.ShapeDtypeStruct((G, 1), F32),
        scratch_shapes=[
            pltpu.VMEM((BR * 128, C), F32),
            pltpu.VMEM((BR * 128, G), F32),
            pltpu.VMEM((G, C), F32),
            pltpu.VMEM((1, G), F32),
        ],
    )(acc.reshape(4, NPAD, 16),
      h1, dinv2, batch2, W2, b2.reshape(1, C), fcW, fcb.reshape(1, 1))

    return out


# speedup vs baseline: 37.1740x; 1.0563x over previous
import functools

import jax
import jax.numpy as jnp
from jax import lax
from jax.experimental import pallas as pl
from jax.experimental.pallas import tpu as pltpu
from jax.experimental.pallas import tpu_sc as plsc

N = 50000
E = 800000
C = 64
G = 8
NPAD = 50176
NROW = 392
EPAD = 802816
NS = 16
CHK = 392
HCHK = CHK // 2
BR = 8
GRID = NROW // BR
F32 = jnp.float32
BF16 = jnp.bfloat16
I32 = jnp.int32

def _zero_vec(ref, nwords):
    @pl.loop(0, nwords // 16)
    def _(i):
        ref[pl.ds(i * 16, 16)] = jnp.zeros((16,), F32)


def _zero_acc_chunks(zbuf, acc_sh, s):
    @pl.loop(0, 25)
    def _(j):
        idx = j * 16 + s

        @pl.when(idx < NROW)
        def _():
            pltpu.sync_copy(zbuf, acc_sh.at[pl.ds(idx * 128, 128)])


def _readout_chunks(acc_sh, out_hbm, c, s):
    @pl.loop(0, 25)
    def _(j):
        idx = j * 16 + s

        @pl.when(idx < NROW)
        def _():
            pltpu.sync_copy(acc_sh.at[pl.ds(idx * 128, 128)],
                            out_hbm.at[pl.ds(c * NPAD + idx * 128, 128)])


@functools.cache
def _sc_kernels():
    mesh = plsc.VectorSubcoreMesh(core_axis_name="c", subcore_axis_name="s")
    params = pltpu.CompilerParams(use_tc_tiling_on_sc=False)

    deg = functools.partial(
        pl.kernel,
        out_type=jax.ShapeDtypeStruct((2 * NPAD,), F32),
        mesh=mesh,
        scratch_types=[
            pltpu.VMEM((128,), F32),
            pltpu.VMEM((128,), F32),
            pltpu.VMEM((HCHK, 128), I32),
            pltpu.VMEM_SHARED((NPAD,), F32),
            pltpu.SemaphoreType.DMA,
        ],
    )(_deg_sc)
    agg1 = functools.partial(
        pl.kernel,
        out_type=jax.ShapeDtypeStruct((2 * NPAD,), F32),
        mesh=mesh,
        scratch_types=[
            pltpu.VMEM((HCHK, 128), I32),
            pltpu.VMEM((HCHK, 128), I32),
            pltpu.VMEM((4, 128), F32),
            pltpu.VMEM((128,), F32),
            pltpu.VMEM_SHARED((NPAD,), F32),
            pltpu.VMEM_SHARED((NPAD,), F32),
            pltpu.SemaphoreType.DMA((4,)),
            pltpu.SemaphoreType.DMA((4,)),
        ],
    )(_agg1_sc)
    agg2 = functools.partial(
        pl.kernel,
        out_type=[jax.ShapeDtypeStruct((NPAD, 16), BF16) for _ in range(4)],
        mesh=mesh,
        scratch_types=[
            pltpu.VMEM((CHK, 128), I32),
            pltpu.VMEM((CHK, 128), I32),
            pltpu.VMEM((4, 128, 16), BF16),
            pltpu.VMEM((128, 16), BF16),
            pltpu.VMEM_SHARED((NPAD, 16), BF16),
            pltpu.SemaphoreType.DMA((4,)),
            pltpu.SemaphoreType.DMA((4,)),
        ],
        compiler_params=params,
    )(_agg2_sc)
    return deg, agg1, agg2


def _deg_sc(col_hbm, out_hbm, ones_v, zbuf, colbuf, acc_sh, ssem):
    c = lax.axis_index("c")
    s = lax.axis_index("s")
    w = s * 2 + c

    @pl.loop(0, 8)
    def _(i):
        ones_v[pl.ds(i * 16, 16)] = jnp.ones((16,), F32)

    _zero_vec(zbuf, 128)
    _zero_acc_chunks(zbuf, acc_sh, s)
    plsc.subcore_barrier()

    pltpu.sync_copy(col_hbm.at[w], colbuf)

    @pl.loop(0, HCHK // 7)
    def _(m):
        for j in range(7):
            pltpu.async_copy(ones_v, acc_sh.at[colbuf.at[m * 7 + j]], ssem,
                             add=True)
        for j in range(7):
            pltpu.make_async_copy(ones_v, acc_sh.at[colbuf.at[m * 7 + j]],
                                  ssem).wait()

    plsc.subcore_barrier()
    _readout_chunks(acc_sh, out_hbm, c, s)


def _prep_tc(degp_ref, x_ref, dinv_ref, u_ref):
    deg = degp_ref[0] + degp_ref[1] + 1.0
    dinv = lax.rsqrt(deg)
    dinv_ref[...] = dinv
    u_ref[...] = x_ref[...] * dinv


def _agg1_sc(row_hbm, col_hbm, u_hbm, out_hbm,
             rowbuf, colbuf, vring, zbuf, u_sh, acc_sh, gsem, ssem):
    c = lax.axis_index("c")
    s = lax.axis_index("s")
    w = s * 2 + c

    _zero_vec(zbuf, 128)
    _zero_acc_chunks(zbuf, acc_sh, s)

    @pl.loop(0, 25)
    def _(j):
        idx = j * 16 + s

        @pl.when(idx < NROW)
        def _():
            pltpu.sync_copy(u_hbm.at[pl.ds(idx * 128, 128)],
                            u_sh.at[pl.ds(idx * 128, 128)])

    plsc.subcore_barrier()

    pltpu.sync_copy(row_hbm.at[w], rowbuf)
    pltpu.sync_copy(col_hbm.at[w], colbuf)

    @pl.loop(0, HCHK // 4)
    def _(m):
        k = m * 4
        for j in range(4):
            @pl.when(m > 0)
            def _():
                pltpu.make_async_copy(
                    vring.at[j], acc_sh.at[colbuf.at[k + j - 4]],
                    ssem.at[j]).wait()

            pltpu.async_copy(u_sh.at[rowbuf.at[k + j]], vring.at[j],
                             gsem.at[j])
        for j in range(4):
            pltpu.make_async_copy(u_sh.at[rowbuf.at[k + j]], vring.at[j],
                                  gsem.at[j]).wait()
            pltpu.async_copy(vring.at[j], acc_sh.at[colbuf.at[k + j]],
                             ssem.at[j], add=True)

    for j in range(4):
        pltpu.make_async_copy(vring.at[j],
                              acc_sh.at[colbuf.at[HCHK - 4 + j]],
                              ssem.at[j]).wait()

    plsc.subcore_barrier()
    _readout_chunks(acc_sh, out_hbm, c, s)


def _h1_tc(x_ref, t_ref, dinv_ref, w1_ref, b1_ref, h1_ref,
           g0_ref, g1_ref, g2_ref, g3_ref):
    g_refs = (g0_ref, g1_ref, g2_ref, g3_ref)
    dinv = dinv_ref[...]
    s1 = dinv * (t_ref[0] + t_ref[1] + x_ref[...] * dinv)
    s1t = jnp.transpose(s1)
    xt = jnp.transpose(x_ref[...])
    dt = jnp.transpose(dinv)
    w1 = w1_ref[...]
    b1 = b1_ref[...]
    for u in range(BR):
        s1c = s1t[:, u:u + 1]
        z = s1c * w1 + b1
        h1 = xt[:, u:u + 1] + z * jax.nn.sigmoid(z)
        g = h1 * dt[:, u:u + 1]
        h1_ref[pl.ds(u * 128, 128), :] = h1
        gb = g.astype(BF16)
        for p in range(4):
            g_refs[p][pl.ds(u * 128, 128), :] = gb[:, p * 16:(p + 1) * 16]


def _agg2_sc(row_hbm, col_hbm, g0_hbm, g1_hbm, g2_hbm, g3_hbm,
             o0_hbm, o1_hbm, o2_hbm, o3_hbm,
             rowbuf, colbuf, gring, zbuf, acc_sh, gsem, ssem):
    c = lax.axis_index("c")
    s = lax.axis_index("s")

    @pl.loop(0, 64)
    def _(r):
        zbuf[pl.ds(r * 2, 2), :] = jnp.zeros((2, 16), BF16)

    pltpu.sync_copy(col_hbm.at[s], colbuf)
    pltpu.sync_copy(row_hbm.at[s], rowbuf)

    def one_pass(g_hbm, o_hbm):
        _zero_acc_chunks(zbuf, acc_sh, s)
        plsc.subcore_barrier()

        @pl.loop(0, CHK // 4)
        def _(m):
            k = m * 4
            for j in range(4):
                @pl.when(m > 0)
                def _():
                    pltpu.make_async_copy(
                        gring.at[j], acc_sh.at[colbuf.at[k + j - 4]],
                        ssem.at[j]).wait()

                pltpu.async_copy(g_hbm.at[rowbuf.at[k + j]], gring.at[j],
                                 gsem.at[j])
            for j in range(4):
                pltpu.make_async_copy(g_hbm.at[rowbuf.at[k + j]], gring.at[j],
                                      gsem.at[j]).wait()
                pltpu.async_copy(gring.at[j], acc_sh.at[colbuf.at[k + j]],
                                 ssem.at[j], add=True)

        for j in range(4):
            pltpu.make_async_copy(gring.at[j],
                                  acc_sh.at[colbuf.at[CHK - 4 + j]],
                                  ssem.at[j]).wait()

        plsc.subcore_barrier()

        @pl.loop(0, 25)
        def _(j):
            idx = j * 16 + s

            @pl.when(idx < NROW)
            def _():
                pltpu.sync_copy(acc_sh.at[pl.ds(idx * 128, 128)],
                                o_hbm.at[pl.ds(idx * 128, 128)])

        plsc.subcore_barrier()

    @pl.when(c == 0)
    def _():
        one_pass(g0_hbm, o0_hbm)
        one_pass(g2_hbm, o2_hbm)

    @pl.when(c == 1)
    def _():
        one_pass(g1_hbm, o1_hbm)
        one_pass(g3_hbm, o3_hbm)


def _fin_tc(acc0_ref, acc1_ref, acc2_ref, acc3_ref,
            h1_ref, dinv_ref, batch_ref, w2_ref, b2_ref,
            fcw_ref, fcb_ref, out_ref, a2s, masks, psum, pcnt):
    acc_refs = (acc0_ref, acc1_ref, acc2_ref, acc3_ref)
    i = pl.program_id(0)

    @pl.when(i == 0)
    def _():
        psum[...] = jnp.zeros((G, C), F32)
        pcnt[...] = jnp.zeros((1, G), F32)

    dt = jnp.transpose(dinv_ref[...])
    bt = jnp.transpose(batch_ref[...])
    h1 = h1_ref[...]
    gid = lax.broadcasted_iota(I32, (128, G), 1)
    for u in range(BR):
        dc = dt[:, u:u + 1]
        accu = jnp.concatenate([r[pl.ds(u * 128, 128), :]
                                for r in acc_refs], axis=1).astype(F32)
        h1u = h1[u * 128:(u + 1) * 128, :]
        a2s[pl.ds(u * 128, 128), :] = dc * accu + (dc * dc) * h1u
        masks[pl.ds(u * 128, 128), :] = (bt[:, u:u + 1] == gid).astype(F32)

    out2 = jnp.dot(a2s[...], w2_ref[...], preferred_element_type=F32)
    out2 = out2 + b2_ref[...]
    h2 = h1 + out2 * jax.nn.sigmoid(out2)
    m = masks[...]
    psum[...] += lax.dot_general(m, h2, (((0,), (0,)), ((), ())),
                                 preferred_element_type=F32)
    pcnt[...] += jnp.sum(m, axis=0, keepdims=True)

    @pl.when(i == GRID - 1)
    def _():
        cnt = jnp.maximum(pcnt[...], 1.0)
        pooled = psum[...] / jnp.transpose(cnt)
        z = jnp.dot(pooled, fcw_ref[...], preferred_element_type=F32)
        out_ref[...] = jax.nn.sigmoid(z + fcb_ref[...])


def kernel(x, edge_index, batch, W1, b1, W2, b2, fcW, fcb):
    ei = edge_index.astype(I32)
    rowf = jnp.concatenate([ei[0], jnp.zeros((EPAD - E,), I32)])
    colf = jnp.concatenate([ei[1], jnp.full((EPAD - E,), N, I32)])
    row_t = rowf.reshape(NS, CHK, 128)
    col_t = colf.reshape(NS, CHK, 128)
    row_w = rowf.reshape(2 * NS, HCHK, 128)
    col_w = colf.reshape(2 * NS, HCHK, 128)
    x2 = jnp.concatenate([x[:, 0], jnp.zeros((NPAD - N,), F32)]).reshape(NROW, 128)
    batch2 = jnp.concatenate([batch.astype(I32),
                              jnp.full((NPAD - N,), 127, I32)]).reshape(NROW, 128)

    deg_k, agg1_k, agg2_k = _sc_kernels()
    degp = deg_k(col_w).reshape(2, NROW, 128)

    dinv2, u2 = pl.pallas_call(
        _prep_tc,
        out_shape=[jax.ShapeDtypeStruct((NROW, 128), F32),
                   jax.ShapeDtypeStruct((NROW, 128), F32)],
    )(degp, x2)

    t = agg1_k(row_w, col_w, u2.reshape(NPAD)).reshape(2, NROW, 128)

    h1, g0, g1, g2, g3 = pl.pallas_call(
        _h1_tc,
        grid=(GRID,),
        in_specs=[
            pl.BlockSpec((BR, 128), lambda i: (i, 0)),
            pl.BlockSpec((2, BR, 128), lambda i: (0, i, 0)),
            pl.BlockSpec((BR, 128), lambda i: (i, 0)),
            pl.BlockSpec((1, C), lambda i: (0, 0)),
            pl.BlockSpec((1, C), lambda i: (0, 0)),
        ],
        out_specs=[pl.BlockSpec((BR * 128, C), lambda i: (i, 0))] +
                  [pl.BlockSpec((BR * 128, 16), lambda i: (i, 0))
                   for _ in range(4)],
        out_shape=[jax.ShapeDtypeStruct((NPAD, C), F32)] +
                  [jax.ShapeDtypeStruct((NPAD, 16), BF16) for _ in range(4)],
    )(x2, t, dinv2, W1, b1.reshape(1, C))

    acc0, acc1, acc2, acc3 = agg2_k(row_t, col_t, g0, g1, g2, g3)

    out = pl.pallas_call(
        _fin_tc,
        grid=(GRID,),
        in_specs=[pl.BlockSpec((BR * 128, 16), lambda i: (i, 0))
                  for _ in range(4)] + [
            pl.BlockSpec((BR * 128, C), lambda i: (i, 0)),
            pl.BlockSpec((BR, 128), lambda i: (i, 0)),
            pl.BlockSpec((BR, 128), lambda i: (i, 0)),
            pl.BlockSpec((C, C), lambda i: (0, 0)),
            pl.BlockSpec((1, C), lambda i: (0, 0)),
            pl.BlockSpec((C, 1), lambda i: (0, 0)),
            pl.BlockSpec((1, 1), lambda i: (0, 0)),
        ],
        out_specs=pl.BlockSpec((G, 1), lambda i: (0, 0)),
        out_shape=jax.ShapeDtypeStruct((G, 1), F32),
        scratch_shapes=[
            pltpu.VMEM((BR * 128, C), F32),
            pltpu.VMEM((BR * 128, G), F32),
            pltpu.VMEM((G, C), F32),
            pltpu.VMEM((1, G), F32),
        ],
    )(acc0, acc1, acc2, acc3,
      h1, dinv2, batch2, W2, b2.reshape(1, C), fcW, fcb.reshape(1, 1))

    return out


# speedup vs baseline: 38.4289x; 1.0338x over previous
import functools

import jax
import jax.numpy as jnp
from jax import lax
from jax.experimental import pallas as pl
from jax.experimental.pallas import tpu as pltpu
from jax.experimental.pallas import tpu_sc as plsc

N = 50000
E = 800000
C = 64
G = 8
NPAD = 50176
NROW = 392
EPAD = 802816
NS = 16
NTILE = NPAD // NS
CHK = 392
HCHK = CHK // 2
BR = 8
GRID = NROW // BR
F32 = jnp.float32
BF16 = jnp.bfloat16
I32 = jnp.int32

def _zero_vec(ref, nwords):
    @pl.loop(0, nwords // 16)
    def _(i):
        ref[pl.ds(i * 16, 16)] = jnp.zeros((16,), F32)


def _zero_acc_chunks(zbuf, acc_sh, s):
    @pl.loop(0, 25)
    def _(j):
        idx = j * 16 + s

        @pl.when(idx < NROW)
        def _():
            pltpu.sync_copy(zbuf, acc_sh.at[pl.ds(idx * 128, 128)])


def _readout_chunks(acc_sh, out_hbm, c, s):
    @pl.loop(0, 25)
    def _(j):
        idx = j * 16 + s

        @pl.when(idx < NROW)
        def _():
            pltpu.sync_copy(acc_sh.at[pl.ds(idx * 128, 128)],
                            out_hbm.at[pl.ds(c * NPAD + idx * 128, 128)])


@functools.cache
def _sc_kernels():
    mesh = plsc.VectorSubcoreMesh(core_axis_name="c", subcore_axis_name="s")
    params = pltpu.CompilerParams(use_tc_tiling_on_sc=False)
    params_nl = pltpu.CompilerParams(needs_layout_passes=False)

    pre = functools.partial(
        pl.kernel,
        out_type=[jax.ShapeDtypeStruct((2 * NPAD,), F32),
                  jax.ShapeDtypeStruct((NPAD,), F32)],
        mesh=mesh,
        scratch_types=[
            pltpu.VMEM((HCHK, 128), I32),
            pltpu.VMEM((CHK, 128), I32),
            pltpu.VMEM((4, 128), F32),
            pltpu.VMEM((128,), F32),
            pltpu.VMEM((128,), F32),
            pltpu.VMEM((NTILE,), F32),
            pltpu.VMEM((NTILE,), F32),
            pltpu.VMEM((NTILE,), F32),
            pltpu.VMEM_SHARED((NPAD,), F32),
            pltpu.VMEM_SHARED((NPAD,), F32),
            pltpu.SemaphoreType.DMA((4,)),
            pltpu.SemaphoreType.DMA((4,)),
        ],
        compiler_params=params_nl,
    )(_pre_sc)
    agg2 = functools.partial(
        pl.kernel,
        out_type=[jax.ShapeDtypeStruct((NPAD, 16), BF16) for _ in range(4)],
        mesh=mesh,
        scratch_types=[
            pltpu.VMEM((CHK, 128), I32),
            pltpu.VMEM((CHK, 128), I32),
            pltpu.VMEM((4, 128, 16), BF16),
            pltpu.VMEM((128, 16), BF16),
            pltpu.VMEM_SHARED((NPAD, 16), BF16),
            pltpu.SemaphoreType.DMA((4,)),
            pltpu.SemaphoreType.DMA((4,)),
        ],
        compiler_params=params,
    )(_agg2_sc)
    return pre, agg2


def _pre_sc(row_hbm, col_hbm, x_hbm, out_hbm, dinv_hbm,
            rowbuf, colbuf, vring, zbuf, ones_v, d0b, xb, db,
            u_sh, acc_sh, gsem, ssem):
    c = lax.axis_index("c")
    s = lax.axis_index("s")
    w = s * 2 + c
    base = s * NTILE
    co = c * HCHK

    @pl.loop(0, 8)
    def _(i):
        ones_v[pl.ds(i * 16, 16)] = jnp.ones((16,), F32)

    _zero_vec(zbuf, 128)
    _zero_acc_chunks(zbuf, acc_sh, s)
    plsc.subcore_barrier()

    pltpu.sync_copy(col_hbm.at[s], colbuf)

    @pl.loop(0, CHK // 7)
    def _(m):
        for j in range(7):
            pltpu.async_copy(ones_v, acc_sh.at[colbuf.at[m * 7 + j]],
                             gsem.at[0], add=True)
        for j in range(7):
            pltpu.make_async_copy(ones_v, acc_sh.at[colbuf.at[m * 7 + j]],
                                  gsem.at[0]).wait()

    plsc.subcore_barrier()

    pltpu.sync_copy(acc_sh.at[pl.ds(base, NTILE)], d0b)
    pltpu.sync_copy(x_hbm.at[pl.ds(base, NTILE)], xb)

    @pl.loop(0, NTILE // 16)
    def _(i):
        sl = pl.ds(i * 16, 16)
        d = d0b[sl] + 1.0
        ibits = plsc.bitcast(d, I32)
        y = plsc.bitcast(jnp.int32(0x5F3759DF) - (ibits >> 1), F32)
        hd = 0.5 * d
        y = y * (1.5 - hd * y * y)
        y = y * (1.5 - hd * y * y)
        y = y * (1.5 - hd * y * y)
        db[sl] = y
        xb[sl] = xb[sl] * y

    pltpu.sync_copy(xb, u_sh.at[pl.ds(base, NTILE)])

    @pl.when(c == 0)
    def _():
        pltpu.sync_copy(db, dinv_hbm.at[pl.ds(base, NTILE)])

    plsc.subcore_barrier()

    _zero_acc_chunks(zbuf, acc_sh, s)
    plsc.subcore_barrier()

    pltpu.sync_copy(row_hbm.at[w], rowbuf)

    @pl.loop(0, HCHK // 4)
    def _(m):
        k = m * 4
        for j in range(4):
            @pl.when(m > 0)
            def _():
                pltpu.make_async_copy(
                    vring.at[j], acc_sh.at[colbuf.at[co + k + j - 4]],
                    ssem.at[j]).wait()

            pltpu.async_copy(u_sh.at[rowbuf.at[k + j]], vring.at[j],
                             gsem.at[j])
        for j in range(4):
            pltpu.make_async_copy(u_sh.at[rowbuf.at[k + j]], vring.at[j],
                                  gsem.at[j]).wait()
            pltpu.async_copy(vring.at[j], acc_sh.at[colbuf.at[co + k + j]],
                             ssem.at[j], add=True)

    for j in range(4):
        pltpu.make_async_copy(vring.at[j],
                              acc_sh.at[colbuf.at[co + HCHK - 4 + j]],
                              ssem.at[j]).wait()

    plsc.subcore_barrier()
    _readout_chunks(acc_sh, out_hbm, c, s)


def _h1_tc(x_ref, t_ref, dinv_ref, w1_ref, b1_ref, h1_ref,
           g0_ref, g1_ref, g2_ref, g3_ref):
    g_refs = (g0_ref, g1_ref, g2_ref, g3_ref)
    dinv = dinv_ref[...]
    s1 = dinv * (t_ref[0] + t_ref[1] + x_ref[...] * dinv)
    s1t = jnp.transpose(s1)
    xt = jnp.transpose(x_ref[...])
    dt = jnp.transpose(dinv)
    w1 = w1_ref[...]
    b1 = b1_ref[...]
    for u in range(BR):
        s1c = s1t[:, u:u + 1]
        z = s1c * w1 + b1
        h1 = xt[:, u:u + 1] + z * jax.nn.sigmoid(z)
        g = h1 * dt[:, u:u + 1]
        h1_ref[pl.ds(u * 128, 128), :] = h1
        gb = g.astype(BF16)
        for p in range(4):
            g_refs[p][pl.ds(u * 128, 128), :] = gb[:, p * 16:(p + 1) * 16]


def _agg2_sc(row_hbm, col_hbm, g0_hbm, g1_hbm, g2_hbm, g3_hbm,
             o0_hbm, o1_hbm, o2_hbm, o3_hbm,
             rowbuf, colbuf, gring, zbuf, acc_sh, gsem, ssem):
    c = lax.axis_index("c")
    s = lax.axis_index("s")

    @pl.loop(0, 64)
    def _(r):
        zbuf[pl.ds(r * 2, 2), :] = jnp.zeros((2, 16), BF16)

    pltpu.sync_copy(col_hbm.at[s], colbuf)
    pltpu.sync_copy(row_hbm.at[s], rowbuf)

    def one_pass(g_hbm, o_hbm):
        _zero_acc_chunks(zbuf, acc_sh, s)
        plsc.subcore_barrier()

        @pl.loop(0, CHK // 4)
        def _(m):
            k = m * 4
            for j in range(4):
                @pl.when(m > 0)
                def _():
                    pltpu.make_async_copy(
                        gring.at[j], acc_sh.at[colbuf.at[k + j - 4]],
                        ssem.at[j]).wait()

                pltpu.async_copy(g_hbm.at[rowbuf.at[k + j]], gring.at[j],
                                 gsem.at[j])
            for j in range(4):
                pltpu.make_async_copy(g_hbm.at[rowbuf.at[k + j]], gring.at[j],
                                      gsem.at[j]).wait()
                pltpu.async_copy(gring.at[j], acc_sh.at[colbuf.at[k + j]],
                                 ssem.at[j], add=True)

        for j in range(4):
            pltpu.make_async_copy(gring.at[j],
                                  acc_sh.at[colbuf.at[CHK - 4 + j]],
                                  ssem.at[j]).wait()

        plsc.subcore_barrier()

        @pl.loop(0, 25)
        def _(j):
            idx = j * 16 + s

            @pl.when(idx < NROW)
            def _():
                pltpu.sync_copy(acc_sh.at[pl.ds(idx * 128, 128)],
                                o_hbm.at[pl.ds(idx * 128, 128)])

        plsc.subcore_barrier()

    @pl.when(c == 0)
    def _():
        one_pass(g0_hbm, o0_hbm)
        one_pass(g2_hbm, o2_hbm)

    @pl.when(c == 1)
    def _():
        one_pass(g1_hbm, o1_hbm)
        one_pass(g3_hbm, o3_hbm)


def _fin_tc(acc0_ref, acc1_ref, acc2_ref, acc3_ref,
            h1_ref, dinv_ref, batch_ref, w2_ref, b2_ref,
            fcw_ref, fcb_ref, out_ref, a2s, masks, psum, pcnt):
    acc_refs = (acc0_ref, acc1_ref, acc2_ref, acc3_ref)
    i = pl.program_id(0)

    @pl.when(i == 0)
    def _():
        psum[...] = jnp.zeros((G, C), F32)
        pcnt[...] = jnp.zeros((1, G), F32)

    dt = jnp.transpose(dinv_ref[...])
    bt = jnp.transpose(batch_ref[...])
    h1 = h1_ref[...]
    gid = lax.broadcasted_iota(I32, (128, G), 1)
    for u in range(BR):
        dc = dt[:, u:u + 1]
        accu = jnp.concatenate([r[pl.ds(u * 128, 128), :]
                                for r in acc_refs], axis=1).astype(F32)
        h1u = h1[u * 128:(u + 1) * 128, :]
        a2s[pl.ds(u * 128, 128), :] = dc * accu + (dc * dc) * h1u
        masks[pl.ds(u * 128, 128), :] = (bt[:, u:u + 1] == gid).astype(F32)

    out2 = jnp.dot(a2s[...], w2_ref[...], preferred_element_type=F32)
    out2 = out2 + b2_ref[...]
    h2 = h1 + out2 * jax.nn.sigmoid(out2)
    m = masks[...]
    psum[...] += lax.dot_general(m, h2, (((0,), (0,)), ((), ())),
                                 preferred_element_type=F32)
    pcnt[...] += jnp.sum(m, axis=0, keepdims=True)

    @pl.when(i == GRID - 1)
    def _():
        cnt = jnp.maximum(pcnt[...], 1.0)
        pooled = psum[...] / jnp.transpose(cnt)
        z = jnp.dot(pooled, fcw_ref[...], preferred_element_type=F32)
        out_ref[...] = jax.nn.sigmoid(z + fcb_ref[...])


def kernel(x, edge_index, batch, W1, b1, W2, b2, fcW, fcb):
    ei = edge_index.astype(I32)
    rowf = jnp.concatenate([ei[0], jnp.zeros((EPAD - E,), I32)])
    colf = jnp.concatenate([ei[1], jnp.full((EPAD - E,), N, I32)])
    row_t = rowf.reshape(NS, CHK, 128)
    col_t = colf.reshape(NS, CHK, 128)
    row_w = rowf.reshape(2 * NS, HCHK, 128)
    col_w = colf.reshape(2 * NS, HCHK, 128)
    x2 = jnp.concatenate([x[:, 0], jnp.zeros((NPAD - N,), F32)]).reshape(NROW, 128)
    batch2 = jnp.concatenate([batch.astype(I32),
                              jnp.full((NPAD - N,), 127, I32)]).reshape(NROW, 128)

    pre_k, agg2_k = _sc_kernels()
    tflat, dinvflat = pre_k(row_w, col_t, x2.reshape(NPAD))
    t = tflat.reshape(2, NROW, 128)
    dinv2 = dinvflat.reshape(NROW, 128)

    h1, g0, g1, g2, g3 = pl.pallas_call(
        _h1_tc,
        grid=(GRID,),
        in_specs=[
            pl.BlockSpec((BR, 128), lambda i: (i, 0)),
            pl.BlockSpec((2, BR, 128), lambda i: (0, i, 0)),
            pl.BlockSpec((BR, 128), lambda i: (i, 0)),
            pl.BlockSpec((1, C), lambda i: (0, 0)),
            pl.BlockSpec((1, C), lambda i: (0, 0)),
        ],
        out_specs=[pl.BlockSpec((BR * 128, C), lambda i: (i, 0))] +
                  [pl.BlockSpec((BR * 128, 16), lambda i: (i, 0))
                   for _ in range(4)],
        out_shape=[jax.ShapeDtypeStruct((NPAD, C), F32)] +
                  [jax.ShapeDtypeStruct((NPAD, 16), BF16) for _ in range(4)],
    )(x2, t, dinv2, W1, b1.reshape(1, C))

    acc0, acc1, acc2, acc3 = agg2_k(row_t, col_t, g0, g1, g2, g3)

    out = pl.pallas_call(
        _fin_tc,
        grid=(GRID,),
        in_specs=[pl.BlockSpec((BR * 128, 16), lambda i: (i, 0))
                  for _ in range(4)] + [
            pl.BlockSpec((BR * 128, C), lambda i: (i, 0)),
            pl.BlockSpec((BR, 128), lambda i: (i, 0)),
            pl.BlockSpec((BR, 128), lambda i: (i, 0)),
            pl.BlockSpec((C, C), lambda i: (0, 0)),
            pl.BlockSpec((1, C), lambda i: (0, 0)),
            pl.BlockSpec((C, 1), lambda i: (0, 0)),
            pl.BlockSpec((1, 1), lambda i: (0, 0)),
        ],
        out_specs=pl.BlockSpec((G, 1), lambda i: (0, 0)),
        out_shape=jax.ShapeDtypeStruct((G, 1), F32),
        scratch_shapes=[
            pltpu.VMEM((BR * 128, C), F32),
            pltpu.VMEM((BR * 128, G), F32),
            pltpu.VMEM((G, C), F32),
            pltpu.VMEM((1, G), F32),
        ],
    )(acc0, acc1, acc2, acc3,
      h1, dinv2, batch2, W2, b2.reshape(1, C), fcW, fcb.reshape(1, 1))

    return out
